# Initial kernel scaffold; baseline (speedup 1.0000x reference)
import functools

import jax
import jax.numpy as jnp
from jax import lax
from jax.experimental import pallas as pl
from jax.experimental.pallas import tpu as pltpu
from jax.experimental.pallas import tpu_sc as plsc

N = 10000
D = 256
DH = 128
E = 160000
NS = 16
CHUNK = 80
CPT = 125
RPT = N // NS
ZROWS = 125


def _row_spec(block_rows, cols):
    return pl.BlockSpec((block_rows, cols), lambda i: (i, 0))


def _rep_spec(r, c):
    return pl.BlockSpec((r, c), lambda i: (0, 0))


def _act(pre_m, pre_v):
    mean = jnp.where(pre_m > 0, pre_m, jnp.expm1(pre_m))
    var = jnp.maximum(pre_v, 0.0)
    att = jnp.exp(-var)
    return mean * att, var * (att * att)


def _dense0_body(x_ref, w_ref, b_ref, m0_ref, m1_ref, v0_ref, v1_ref):
    pre = lax.dot_general(x_ref[...], w_ref[...], (((1,), (1,)), ((), ())),
                          preferred_element_type=jnp.float32) + b_ref[...]
    me, ve = _act(pre, pre)
    m0_ref[...] = me[:, :DH]
    m1_ref[...] = me[:, DH:]
    v0_ref[...] = ve[:, :DH]
    v1_ref[...] = ve[:, DH:]


def _dense1_body(m0_ref, m1_ref, v0_ref, v1_ref, wm_ref, bm_ref, wv_ref,
                 bv_ref, om0_ref, om1_ref, ov0_ref, ov1_ref):
    m = jnp.concatenate([m0_ref[...], m1_ref[...]], axis=1)
    v = jnp.concatenate([v0_ref[...], v1_ref[...]], axis=1)
    pre_m = lax.dot_general(m, wm_ref[...], (((1,), (1,)), ((), ())),
                            preferred_element_type=jnp.float32) + bm_ref[...]
    pre_v = lax.dot_general(v, wv_ref[...], (((1,), (1,)), ((), ())),
                            preferred_element_type=jnp.float32) + bv_ref[...]
    me, ve = _act(pre_m, pre_v)
    om0_ref[...] = me[:, :DH]
    om1_ref[...] = me[:, DH:]
    ov0_ref[...] = ve[:, :DH]
    ov1_ref[...] = ve[:, DH:]


def _final_body(m0_ref, m1_ref, v0_ref, v1_ref, s_ref, o_ref):
    m = jnp.concatenate([m0_ref[...], m1_ref[...]], axis=1)
    v = jnp.concatenate([v0_ref[...], v1_ref[...]], axis=1)
    o_ref[...] = m + s_ref[...] * jnp.sqrt(v)


_BR = 1000

_half = jax.ShapeDtypeStruct((N, DH), jnp.float32)

_dense0 = pl.pallas_call(
    _dense0_body,
    grid=(N // _BR,),
    in_specs=[_row_spec(_BR, D), _rep_spec(D, D), _rep_spec(1, D)],
    out_specs=[_row_spec(_BR, DH)] * 4,
    out_shape=[_half] * 4,
)

_dense1 = pl.pallas_call(
    _dense1_body,
    grid=(N // _BR,),
    in_specs=[_row_spec(_BR, DH)] * 4
    + [_rep_spec(D, D), _rep_spec(1, D), _rep_spec(D, D), _rep_spec(1, D)],
    out_specs=[_row_spec(_BR, DH)] * 4,
    out_shape=[_half] * 4,
)

_final = pl.pallas_call(
    _final_body,
    grid=(N // _BR,),
    in_specs=[_row_spec(_BR, DH)] * 4 + [_row_spec(_BR, D)],
    out_specs=_row_spec(_BR, D),
    out_shape=jax.ShapeDtypeStruct((N, D), jnp.float32),
)


def _spmm_body(hm0, hm1, hv0, hv1, src2d, dst2d, wm2d, wv2d,
               om0, om1, ov0, ov1,
               src_v, dst_v, w_v, rows_v, zeros_v, acc, sem):
    c = lax.axis_index("c")
    s = lax.axis_index("s")

    zvec = jnp.zeros((16,), jnp.float32)

    def _zrow(i, carry):
        for j in range(DH // 16):
            zeros_v[i, pl.ds(j * 16, 16)] = zvec
        return carry

    lax.fori_loop(0, ZROWS, _zrow, 0)

    pltpu.sync_copy(src2d.at[pl.ds(s * CPT, CPT)], src_v)
    pltpu.sync_copy(dst2d.at[pl.ds(s * CPT, CPT)], dst_v)

    def _phase(h0, h1, w2d, o0, o1):
        for i in range(RPT // ZROWS):
            pltpu.sync_copy(zeros_v, acc.at[pl.ds(s * RPT + i * ZROWS, ZROWS)])
        pltpu.sync_copy(w2d.at[pl.ds(s * CPT, CPT)], w_v)
        plsc.subcore_barrier()

        def _chunk(k, carry):
            @pl.when(c == 0)
            def _():
                pltpu.async_copy(h0.at[src_v.at[k]], rows_v, sem).wait()

            @pl.when(c == 1)
            def _():
                pltpu.async_copy(h1.at[src_v.at[k]], rows_v, sem).wait()

            def _edge(e, ecarry):
                wb = plsc.load_gather(
                    w_v, [jnp.full((16,), k, jnp.int32),
                          jnp.full((16,), e, jnp.int32)])
                for j in range(DH // 16):
                    sl = (e, pl.ds(j * 16, 16))
                    rows_v[sl] = rows_v[sl] * wb
                return ecarry

            lax.fori_loop(0, CHUNK, _edge, 0)
            pltpu.sync_copy(rows_v, acc.at[dst_v.at[k]], add=True)
            return carry

        lax.fori_loop(0, CPT, _chunk, 0)
        plsc.subcore_barrier()

        @pl.when(c == 0)
        def _():
            pltpu.sync_copy(acc.at[pl.ds(s * RPT, RPT)],
                            o0.at[pl.ds(s * RPT, RPT)])

        @pl.when(c == 1)
        def _():
            pltpu.sync_copy(acc.at[pl.ds(s * RPT, RPT)],
                            o1.at[pl.ds(s * RPT, RPT)])

        plsc.subcore_barrier()

    _phase(hm0, hm1, wm2d, om0, om1)
    _phase(hv0, hv1, wv2d, ov0, ov1)


_spmm = functools.partial(
    pl.kernel,
    out_type=[_half] * 4,
    mesh=plsc.VectorSubcoreMesh(core_axis_name="c", subcore_axis_name="s"),
    scratch_types=[
        pltpu.VMEM((CPT, CHUNK), jnp.int32),
        pltpu.VMEM((CPT, CHUNK), jnp.int32),
        pltpu.VMEM((CPT, CHUNK), jnp.float32),
        pltpu.VMEM((CHUNK, DH), jnp.float32),
        pltpu.VMEM((ZROWS, DH), jnp.float32),
        pltpu.VMEM_SHARED((N, DH), jnp.float32),
        pltpu.SemaphoreType.DMA,
    ],
)(_spmm_body)


def kernel(x, edge_index, adj0_w, adj1_w, Wm0, bm0, Wm1, bm1, Wv1, bv1):
    src = edge_index[0].astype(jnp.int32).reshape(E // CHUNK, CHUNK)
    dst = edge_index[1].astype(jnp.int32).reshape(E // CHUNK, CHUNK)
    wm2 = adj0_w.reshape(E // CHUNK, CHUNK)
    wv2 = adj1_w.reshape(E // CHUNK, CHUNK)
    bm0r = bm0.reshape(1, D)
    bm1r = bm1.reshape(1, D)
    bv1r = bv1.reshape(1, D)

    hm0, hm1, hv0, hv1 = _dense0(x, Wm0, bm0r)
    m0, m1, v0, v1 = _spmm(hm0, hm1, hv0, hv1, src, dst, wm2, wv2)
    hm0, hm1, hv0, hv1 = _dense1(m0, m1, v0, v1, Wm1, bm1r, Wv1, bv1r)
    m0, m1, v0, v1 = _spmm(hm0, hm1, hv0, hv1, src, dst, wm2, wv2)

    sample = jax.random.normal(jax.random.key(42), (N, D), jnp.float32)
    return _final(m0, m1, v0, v1, sample)


# speedup vs baseline: 1.7817x; 1.7817x over previous
import functools

import jax
import jax.numpy as jnp
from jax import lax
from jax.experimental import pallas as pl
from jax.experimental.pallas import tpu as pltpu
from jax.experimental.pallas import tpu_sc as plsc

N = 10000
NP = 10240
D = 256
DH = 128
E = 160000
EP = 163840
NS = 16
CHUNK = 80
CPT = 128
RPT = NP // NS
ZROWS = 32
SLAB = 16


def _row_spec(block_rows, cols):
    return pl.BlockSpec((block_rows, cols), lambda i: (i, 0))


def _rep_spec(r, c):
    return pl.BlockSpec((r, c), lambda i: (0, 0))


def _act(pre_m, pre_v):
    mean = jnp.where(pre_m > 0, pre_m, jnp.exp(jnp.minimum(pre_m, 0.0)) - 1.0)
    var = jnp.maximum(pre_v, 0.0)
    att = jnp.exp(-var)
    return mean * att, var * (att * att)


def _dense0_body(x_ref, w_ref, b_ref, m0_ref, m1_ref, v0_ref, v1_ref):
    pre = lax.dot_general(x_ref[...], w_ref[...], (((1,), (1,)), ((), ())),
                          preferred_element_type=jnp.float32) + b_ref[...]
    me, ve = _act(pre, pre)
    m0_ref[...] = me[:, :DH]
    m1_ref[...] = me[:, DH:]
    v0_ref[...] = ve[:, :DH]
    v1_ref[...] = ve[:, DH:]


def _dense1_body(m0_ref, m1_ref, v0_ref, v1_ref, wm_ref, bm_ref, wv_ref,
                 bv_ref, om0_ref, om1_ref, ov0_ref, ov1_ref):
    m = jnp.concatenate([m0_ref[...], m1_ref[...]], axis=1)
    v = jnp.concatenate([v0_ref[...], v1_ref[...]], axis=1)
    pre_m = lax.dot_general(m, wm_ref[...], (((1,), (1,)), ((), ())),
                            preferred_element_type=jnp.float32) + bm_ref[...]
    pre_v = lax.dot_general(v, wv_ref[...], (((1,), (1,)), ((), ())),
                            preferred_element_type=jnp.float32) + bv_ref[...]
    me, ve = _act(pre_m, pre_v)
    om0_ref[...] = me[:, :DH]
    om1_ref[...] = me[:, DH:]
    ov0_ref[...] = ve[:, :DH]
    ov1_ref[...] = ve[:, DH:]


def _final_body(m0_ref, m1_ref, v0_ref, v1_ref, s_ref, o_ref):
    m = jnp.concatenate([m0_ref[...], m1_ref[...]], axis=1)
    v = jnp.concatenate([v0_ref[...], v1_ref[...]], axis=1)
    o_ref[...] = m + s_ref[...] * jnp.sqrt(v)


_BR = 1024

_half = jax.ShapeDtypeStruct((NP, DH), jnp.float32)

_dense0 = pl.pallas_call(
    _dense0_body,
    grid=(NP // _BR,),
    in_specs=[_row_spec(_BR, D), _rep_spec(D, D), _rep_spec(1, D)],
    out_specs=[_row_spec(_BR, DH)] * 4,
    out_shape=[_half] * 4,
)

_dense1 = pl.pallas_call(
    _dense1_body,
    grid=(NP // _BR,),
    in_specs=[_row_spec(_BR, DH)] * 4
    + [_rep_spec(D, D), _rep_spec(1, D), _rep_spec(D, D), _rep_spec(1, D)],
    out_specs=[_row_spec(_BR, DH)] * 4,
    out_shape=[_half] * 4,
)

_final = pl.pallas_call(
    _final_body,
    grid=(NP // _BR,),
    in_specs=[_row_spec(_BR, DH)] * 4 + [_row_spec(_BR, D)],
    out_specs=_row_spec(_BR, D),
    out_shape=jax.ShapeDtypeStruct((NP, D), jnp.float32),
)


def _spmm_body(hm0, hm1, hv0, hv1, src2d, dst2d, wm2d, wv2d,
               om0, om1, ov0, ov1,
               src_v, dst_v, w_v, rows_v, zeros_v, acc, sem):
    c = lax.axis_index("c")
    s = lax.axis_index("s")

    zvec = jnp.zeros((16,), jnp.float32)

    for i in range(ZROWS):
        for j in range(DH // 16):
            zeros_v[i, pl.ds(j * 16, 16)] = zvec

    def _phase(h0, h1, w2d, o0, o1):
        def _zcp(i, zc):
            pltpu.sync_copy(zeros_v, acc.at[pl.ds(s * RPT + i * ZROWS, ZROWS)])
            return zc
        lax.fori_loop(0, RPT // ZROWS, _zcp, 0)
        plsc.subcore_barrier()

        def _slab(b, carry):
            base = s * CPT + b * SLAB
            pltpu.sync_copy(src2d.at[pl.ds(base, SLAB)], src_v)
            pltpu.sync_copy(dst2d.at[pl.ds(base, SLAB)], dst_v)
            pltpu.sync_copy(w2d.at[pl.ds(base, SLAB)], w_v)

            def _chunk(k, kcarry):
                @pl.when(c == 0)
                def _():
                    pltpu.async_copy(h0.at[src_v.at[k]], rows_v, sem).wait()

                @pl.when(c == 1)
                def _():
                    pltpu.async_copy(h1.at[src_v.at[k]], rows_v, sem).wait()

                def _group(g, gcarry):
                    wg = w_v[k, pl.ds(g * 16, 16)]
                    for l in range(16):
                        e = g * 16 + l
                        wb = jnp.full((16,), wg[l], jnp.float32)
                        for j in range(DH // 16):
                            sl = (e, pl.ds(j * 16, 16))
                            rows_v[sl] = rows_v[sl] * wb
                    return gcarry

                lax.fori_loop(0, CHUNK // 16, _group, 0)
                pltpu.sync_copy(rows_v, acc.at[dst_v.at[k]], add=True)
                return kcarry

            lax.fori_loop(0, SLAB, _chunk, 0)
            return carry

        lax.fori_loop(0, CPT // SLAB, _slab, 0)
        plsc.subcore_barrier()

        @pl.when(c == 0)
        def _():
            pltpu.sync_copy(acc.at[pl.ds(s * RPT, RPT)],
                            o0.at[pl.ds(s * RPT, RPT)])

        @pl.when(c == 1)
        def _():
            pltpu.sync_copy(acc.at[pl.ds(s * RPT, RPT)],
                            o1.at[pl.ds(s * RPT, RPT)])

        plsc.subcore_barrier()

    _phase(hm0, hm1, wm2d, om0, om1)
    _phase(hv0, hv1, wv2d, ov0, ov1)


@functools.cache
def _make_spmm():
    return functools.partial(
        pl.kernel,
        out_type=[_half] * 4,
        mesh=plsc.VectorSubcoreMesh(core_axis_name="c", subcore_axis_name="s"),
        scratch_types=[
            pltpu.VMEM((SLAB, CHUNK), jnp.int32),
            pltpu.VMEM((SLAB, CHUNK), jnp.int32),
            pltpu.VMEM((SLAB, CHUNK), jnp.float32),
            pltpu.VMEM((CHUNK, DH), jnp.float32),
            pltpu.VMEM((ZROWS, DH), jnp.float32),
            pltpu.VMEM_SHARED((NP, DH), jnp.float32),
            pltpu.SemaphoreType.DMA,
        ],
    )(_spmm_body)


def kernel(x, edge_index, adj0_w, adj1_w, Wm0, bm0, Wm1, bm1, Wv1, bv1):
    epad = EP - E
    src = jnp.pad(edge_index[0].astype(jnp.int32), (0, epad)).reshape(
        EP // CHUNK, CHUNK)
    dst = jnp.pad(edge_index[1].astype(jnp.int32), (0, epad)).reshape(
        EP // CHUNK, CHUNK)
    wm2 = jnp.pad(adj0_w, (0, epad)).reshape(EP // CHUNK, CHUNK)
    wv2 = jnp.pad(adj1_w, (0, epad)).reshape(EP // CHUNK, CHUNK)
    xp = jnp.pad(x, ((0, NP - N), (0, 0)))
    bm0r = bm0.reshape(1, D)
    bm1r = bm1.reshape(1, D)
    bv1r = bv1.reshape(1, D)

    spmm = _make_spmm()
    hm0, hm1, hv0, hv1 = _dense0(xp, Wm0, bm0r)
    m0, m1, v0, v1 = spmm(hm0, hm1, hv0, hv1, src, dst, wm2, wv2)
    hm0, hm1, hv0, hv1 = _dense1(m0, m1, v0, v1, Wm1, bm1r, Wv1, bv1r)
    m0, m1, v0, v1 = spmm(hm0, hm1, hv0, hv1, src, dst, wm2, wv2)

    sample = jax.random.normal(jax.random.key(42), (N, D), jnp.float32)
    sp = jnp.pad(sample, ((0, NP - N), (0, 0)))
    return _final(m0, m1, v0, v1, sp)[:N]
